# baseline (speedup 1.0000x reference)
import jax
import jax.numpy as jnp
from jax.experimental import pallas as pl
from jax.experimental.pallas import tpu as pltpu

B, L, D, H = 8, 2048, 768, 8
DH = D // H
ST = 256
NT = L // ST
MW = 128
WIN = ST + 2 * MW
SCALE = 1.0 / (DH ** 0.5)
F32 = jnp.float32
BF16 = jnp.bfloat16


def _dot(a, b):
    return jax.lax.dot_general(a, b, (((1,), (0,)), ((), ())),
                               preferred_element_type=F32)


def _body(x_ref, pb_ref, wq_ref, wk_ref, wv_ref, wo_ref,
          bq_ref, bk_ref, bv_ref, bo_ref, o_ref,
          seg_sc, logits_sc, yv_sc):
    pid = pl.program_id(0)

    @pl.when(pid == 0)
    def _seg_all():
        bnd_all = (jnp.reshape(pb_ref[...], (B, L)) != 0).astype(BF16)
        parts = []
        for t in range(NT):
            tril = (jax.lax.broadcasted_iota(jnp.int32, (L, ST), 0) <=
                    jax.lax.broadcasted_iota(jnp.int32, (L, ST), 1) + t * ST
                    ).astype(BF16)
            parts.append(_dot(bnd_all, tril))
        seg_all = (jnp.concatenate(parts, axis=1) -
                   bnd_all[:, :1].astype(F32))
        seg_sc[...] = seg_all.astype(jnp.int32)

    x = x_ref[0]
    seg_i = seg_sc[pl.ds(pid, 1), :]
    seg_col = jnp.reshape(seg_i, (L, 1))
    maxseg = seg_i[0, L - 1]
    nt_used = maxseg // ST + 1
    short_segs = jnp.all(seg_i[0, MW:] > seg_i[0, :L - MW])

    def win_off(t):
        return min(max(t * ST - MW, 0), L - WIN)

    e_mat = (jax.lax.broadcasted_iota(jnp.int32, (D, H), 0) // DH ==
             jax.lax.broadcasted_iota(jnp.int32, (D, H), 1)).astype(BF16)

    xq = _dot(x, wq_ref[...]).astype(BF16)
    xaug = jnp.concatenate([xq, jnp.ones((L, 1), BF16)], axis=1)

    def pass1(offs, width):
        for t in range(NT):
            sl = slice(t * ST, (t + 1) * ST)
            csl = slice(offs[t], offs[t] + width)
            a = (seg_col[sl] ==
                 jnp.broadcast_to(seg_i[:, csl], (ST, width))).astype(BF16)
            r = _dot(a, xaug[csl])
            qj = r[:, :D] / r[:, D:] + bq_ref[...]
            k_t = _dot(x[sl], wk_ref[...]) + bk_ref[...]
            prod = (qj * k_t).astype(BF16)
            logits_sc[sl] = _dot(prod, e_mat) * SCALE

    def pass2(e, e16, offs, width):
        for t in range(NT):
            sl = slice(t * ST, (t + 1) * ST)
            csl = slice(offs[t], offs[t] + width)
            a = (seg_col[sl] ==
                 jnp.broadcast_to(seg_i[:, csl], (ST, width))).astype(BF16)
            denpos = _dot(a, e16[csl])
            wgt = e[sl] / jnp.maximum(denpos, 1e-30)
            wexp = _dot(wgt.astype(BF16), e_mat.T)
            v_t = _dot(x[sl], wv_ref[...]) + bv_ref[...]
            yv_sc[sl] = (wexp * v_t).astype(BF16)

    woffs = [win_off(t) for t in range(NT)]
    zoffs = [0] * NT

    @pl.when(short_segs)
    def _p1_fast():
        pass1(woffs, WIN)

    @pl.when(jnp.logical_not(short_segs))
    def _p1_full():
        pass1(zoffs, L)

    logits = logits_sc[...]
    m = jnp.max(logits, axis=0, keepdims=True)
    e = jnp.exp(logits - m)
    e16 = e.astype(BF16)

    @pl.when(short_segs)
    def _p2_fast():
        pass2(e, e16, woffs, WIN)

    @pl.when(jnp.logical_not(short_segs))
    def _p2_full():
        pass2(e, e16, zoffs, L)

    yv = yv_sc[...]

    row_id = jax.lax.broadcasted_iota(jnp.int32, (ST, 1), 0)
    for t in range(NT):

        @pl.when(t < nt_used)
        def _store():
            kw = L - t * ST
            i0 = jax.lax.broadcasted_iota(jnp.int32, (ST, kw), 0) + t * ST
            p = (i0 == jnp.broadcast_to(seg_i[:, t * ST:], (ST, kw))
                 ).astype(BF16)
            attn = _dot(p, yv[t * ST:]).astype(BF16)
            out = _dot(attn, wo_ref[...]) + bo_ref[...]
            out = jnp.where(row_id + t * ST <= maxseg, out, 0.0)
            o_ref[0, t * ST:(t + 1) * ST, :] = out

        @pl.when(t >= nt_used)
        def _zero():
            o_ref[0, t * ST:(t + 1) * ST, :] = jnp.zeros((ST, D), F32)


@jax.jit
def kernel(byte_repr, patch_boundaries, Wq, Wk, Wv, bq, bk, bv, Wo, bo):
    pb3 = patch_boundaries.reshape(B, 1, L)
    full = lambda shape: pl.BlockSpec(shape, lambda b: (0,) * len(shape))
    out = pl.pallas_call(
        _body,
        grid=(B,),
        in_specs=[
            pl.BlockSpec((1, L, D), lambda b: (b, 0, 0)),
            full((B, 1, L)),
            full((D, D)), full((D, D)), full((D, D)), full((D, D)),
            full((1, D)), full((1, D)), full((1, D)), full((1, D)),
        ],
        out_specs=pl.BlockSpec((1, L, D), lambda b: (b, 0, 0)),
        out_shape=jax.ShapeDtypeStruct((B, L, D), F32),
        scratch_shapes=[pltpu.VMEM((B, L), jnp.int32),
                        pltpu.VMEM((L, H), F32),
                        pltpu.VMEM((L, D), BF16)],
    )(byte_repr.astype(BF16), pb3,
      Wq.T.astype(BF16), Wk.T.astype(BF16), Wv.T.astype(BF16),
      Wo.T.astype(BF16),
      bq.reshape(1, D), bk.reshape(1, D), bv.reshape(1, D), bo.reshape(1, D))
    return out


# speedup vs baseline: 1.2957x; 1.2957x over previous
import jax
import jax.numpy as jnp
from jax.experimental import pallas as pl
from jax.experimental.pallas import tpu as pltpu

B, L, D, H = 8, 2048, 768, 8
DH = D // H
ST = 256
NT = L // ST
SCALE = 1.0 / (DH ** 0.5)
F32 = jnp.float32
BF16 = jnp.bfloat16


def _dot(a, b):
    return jax.lax.dot_general(a, b, (((1,), (0,)), ((), ())),
                               preferred_element_type=F32)


def _body(x_ref, pb_ref, wq_ref, wk_ref, wv_ref, wo_ref,
          bq_ref, bk_ref, bv_ref, bo_ref, o_ref, seg_sc):
    pid = pl.program_id(0)

    @pl.when(pid == 0)
    def _seg_all():
        bnd_all = (jnp.reshape(pb_ref[...], (B, L)) != 0).astype(BF16)
        parts = []
        for t in range(NT):
            tril = (jax.lax.broadcasted_iota(jnp.int32, (L, ST), 0) <=
                    jax.lax.broadcasted_iota(jnp.int32, (L, ST), 1) + t * ST
                    ).astype(BF16)
            parts.append(_dot(bnd_all, tril))
        seg_all = (jnp.concatenate(parts, axis=1) -
                   bnd_all[:, :1].astype(F32))
        seg_sc[...] = seg_all.astype(jnp.int32)

    x = x_ref[0]
    seg_i = seg_sc[pl.ds(pid, 1), :]
    seg_col = jnp.reshape(seg_i, (L, 1))
    maxseg = seg_i[0, L - 1]
    nt_used = maxseg // ST + 1

    def a_tile(t):
        rows = seg_col[t * ST:(t + 1) * ST]
        return (rows == jnp.broadcast_to(seg_i, (ST, L))).astype(BF16)

    e_mat = (jax.lax.broadcasted_iota(jnp.int32, (D, H), 0) // DH ==
             jax.lax.broadcasted_iota(jnp.int32, (D, H), 1)).astype(BF16)

    xq = _dot(x, wq_ref[...]).astype(BF16)
    xaug = jnp.concatenate([xq, jnp.ones((L, 1), BF16)], axis=1)
    logit_tiles = []
    for t in range(NT):
        sl = slice(t * ST, (t + 1) * ST)
        a = a_tile(t)
        r = _dot(a, xaug)
        qj = r[:, :D] / r[:, D:] + bq_ref[...]
        k_t = _dot(x[sl], wk_ref[...]) + bk_ref[...]
        prod = (qj * k_t).astype(BF16)
        logit_tiles.append(_dot(prod, e_mat) * SCALE)
    logits = jnp.concatenate(logit_tiles, axis=0)
    m = jnp.max(logits, axis=0, keepdims=True)
    e = jnp.exp(logits - m)
    e16 = e.astype(BF16)

    yv_tiles = []
    for t in range(NT):
        sl = slice(t * ST, (t + 1) * ST)
        a = a_tile(t)
        denpos = _dot(a, e16)
        wgt = e[sl] / jnp.maximum(denpos, 1e-30)
        wexp = _dot(wgt.astype(BF16), e_mat.T)
        v_t = _dot(x[sl], wv_ref[...]) + bv_ref[...]
        yv_tiles.append((wexp * v_t).astype(BF16))
    yv = jnp.concatenate(yv_tiles, axis=0)

    row_id = jax.lax.broadcasted_iota(jnp.int32, (ST, 1), 0)
    for t in range(NT):

        @pl.when(t < nt_used)
        def _store():
            kw = L - t * ST
            i0 = jax.lax.broadcasted_iota(jnp.int32, (ST, kw), 0) + t * ST
            p = (i0 == jnp.broadcast_to(seg_i[:, t * ST:], (ST, kw))
                 ).astype(BF16)
            attn = _dot(p, yv[t * ST:]).astype(BF16)
            out = _dot(attn, wo_ref[...]) + bo_ref[...]
            out = jnp.where(row_id + t * ST <= maxseg, out, 0.0)
            o_ref[0, t * ST:(t + 1) * ST, :] = out

        @pl.when(t >= nt_used)
        def _zero():
            o_ref[0, t * ST:(t + 1) * ST, :] = jnp.zeros((ST, D), F32)


@jax.jit
def kernel(byte_repr, patch_boundaries, Wq, Wk, Wv, bq, bk, bv, Wo, bo):
    pb3 = patch_boundaries.reshape(B, 1, L)
    full = lambda shape: pl.BlockSpec(shape, lambda b: (0,) * len(shape))
    out = pl.pallas_call(
        _body,
        grid=(B,),
        in_specs=[
            pl.BlockSpec((1, L, D), lambda b: (b, 0, 0)),
            full((B, 1, L)),
            full((D, D)), full((D, D)), full((D, D)), full((D, D)),
            full((1, D)), full((1, D)), full((1, D)), full((1, D)),
        ],
        out_specs=pl.BlockSpec((1, L, D), lambda b: (b, 0, 0)),
        out_shape=jax.ShapeDtypeStruct((B, L, D), F32),
        scratch_shapes=[pltpu.VMEM((B, L), jnp.int32)],
    )(byte_repr.astype(BF16), pb3,
      Wq.T.astype(BF16), Wk.T.astype(BF16), Wv.T.astype(BF16),
      Wo.T.astype(BF16),
      bq.reshape(1, D), bk.reshape(1, D), bv.reshape(1, D), bo.reshape(1, D))
    return out
